# baseline (speedup 1.0000x reference)
import functools

import jax
import jax.numpy as jnp
from jax import lax
from jax.experimental import pallas as pl
from jax.experimental.pallas import tpu as pltpu, tpu_sc as plsc

NUM_CORES = 2
NUM_SUBCORES = 16
NW = NUM_CORES * NUM_SUBCORES
L = 16
B_TC = 8192
TC_CHUNK = 512
TC_DRAIN = 2048


def _sc_embed(labels2d, table, b_per_w, D):
    mesh = plsc.VectorSubcoreMesh(core_axis_name="c", subcore_axis_name="s")
    n_groups = b_per_w // L

    @functools.partial(
        pl.kernel,
        out_type=jax.ShapeDtypeStruct((NW, b_per_w, D), jnp.float32),
        mesh=mesh,
        scratch_types=[
            pltpu.VMEM((b_per_w,), jnp.int32),
            pltpu.VMEM((b_per_w, D), jnp.float32),
            pltpu.SemaphoreType.DMA,
        ],
    )
    def k(table_hbm, idx_hbm, out_hbm, idx_v, rows_v, sem):
        wid = lax.axis_index("s") * NUM_CORES + lax.axis_index("c")
        pltpu.sync_copy(idx_hbm.at[wid], idx_v)

        def group(g, _):
            vec = idx_v[pl.ds(g * L, L)]
            for l in range(L):
                r = jnp.squeeze(lax.slice(vec, (l,), (l + 1,)))
                pltpu.async_copy(table_hbm.at[r], rows_v.at[g * L + l], sem)
            return 0

        lax.fori_loop(0, n_groups, group, 0)
        pltpu.make_async_copy(out_hbm.at[wid], rows_v, sem).wait()
        pltpu.sync_copy(rows_v, out_hbm.at[wid])

    return k(table, labels2d)


def _tc_embed(labels_tc, table, D):
    n_chunks = B_TC // TC_CHUNK

    def body(idx_hbm, table_hbm, out_hbm, idx_s, buf, sem_l, sem_g):
        def chunk(c, _):
            cp = pltpu.make_async_copy(
                idx_hbm.at[pl.ds(c * TC_CHUNK, TC_CHUNK)], idx_s, sem_l
            )
            cp.start()
            cp.wait()

            def row(i, _):
                r = idx_s[i]
                pltpu.make_async_copy(
                    table_hbm.at[pl.ds(r, 1)],
                    buf.at[pl.ds(c * TC_CHUNK + i, 1)],
                    sem_g,
                ).start()
                return 0

            lax.fori_loop(0, TC_CHUNK, row, 0, unroll=4)
            return 0

        lax.fori_loop(0, n_chunks, chunk, 0)

        def drain(c, _):
            pltpu.make_async_copy(
                table_hbm.at[pl.ds(0, TC_DRAIN)],
                buf.at[pl.ds(c * TC_DRAIN, TC_DRAIN)],
                sem_g,
            ).wait()
            return 0

        lax.fori_loop(0, B_TC // TC_DRAIN, drain, 0)

        cp = pltpu.make_async_copy(buf, out_hbm, sem_l)
        cp.start()
        cp.wait()

    return pl.pallas_call(
        body,
        out_shape=jax.ShapeDtypeStruct((B_TC, D), jnp.float32),
        in_specs=[
            pl.BlockSpec(memory_space=pl.ANY),
            pl.BlockSpec(memory_space=pl.ANY),
        ],
        out_specs=pl.BlockSpec(memory_space=pl.ANY),
        scratch_shapes=[
            pltpu.SMEM((TC_CHUNK,), jnp.int32),
            pltpu.VMEM((B_TC, D), jnp.float32),
            pltpu.SemaphoreType.DMA,
            pltpu.SemaphoreType.DMA,
        ],
    )(labels_tc, table)


def kernel(labels, train, table):
    B = labels.shape[0]
    V, D = table.shape
    lab = labels.astype(jnp.int32)
    b_sc = B - B_TC
    b_per_w = b_sc // NW
    sc_out = _sc_embed(lab[:b_sc].reshape(NW, b_per_w), table, b_per_w, D)
    tc_out = _tc_embed(lab[b_sc:], table, D)
    return jnp.concatenate([sc_out.reshape(b_sc, D), tc_out], axis=0)
